# Initial kernel scaffold; baseline (speedup 1.0000x reference)
import jax
import jax.numpy as jnp
from jax.experimental import pallas as pl


def kernel(x, adj_norm, W):
    raise NotImplementedError("write your pallas kernel here")


# speedup vs baseline: 1.0419x; 1.0419x over previous
import jax
import jax.numpy as jnp
from jax.experimental import pallas as pl
from jax.experimental.pallas import tpu as pltpu

_BM = 400


def _gcn_body(x_ref, w_ref, adj_ref, out_ref, h_ref):
    i = pl.program_id(0)

    @pl.when(i == 0)
    def _project():
        h_ref[...] = jax.lax.dot_general(
            x_ref[...], w_ref[...],
            dimension_numbers=(((1,), (1,)), ((), ())),
            preferred_element_type=jnp.float32,
        )

    out_ref[...] = jnp.dot(
        adj_ref[...], h_ref[...], preferred_element_type=jnp.float32
    )


def kernel(x, adj_norm, W):
    n, d_in = x.shape
    d_out = W.shape[0]
    bm = _BM if n % _BM == 0 else n
    grid = (n // bm,)
    return pl.pallas_call(
        _gcn_body,
        grid=grid,
        in_specs=[
            pl.BlockSpec((n, d_in), lambda i: (0, 0)),
            pl.BlockSpec((d_out, d_in), lambda i: (0, 0)),
            pl.BlockSpec((bm, n), lambda i: (i, 0)),
        ],
        out_specs=pl.BlockSpec((bm, d_out), lambda i: (i, 0)),
        out_shape=jax.ShapeDtypeStruct((n, d_out), jnp.float32),
        scratch_shapes=[pltpu.VMEM((n, d_out), jnp.float32)],
    )(x, W, adj_norm)
